# baseline (speedup 1.0000x reference)
import functools

import jax
from jax import lax
from jax.experimental import pallas as pl
from jax.experimental.pallas import tpu as pltpu
from jax.experimental.pallas import tpu_sc as plsc

_NC, _NS = 2, 16
_CHUNKS = (8, 56, 64)


def kernel(x, table):
    seq_len = x.shape[1]
    d = table.shape[1]
    nw = _NC * _NS
    rows_per_w = seq_len // nw
    assert sum(_CHUNKS) == rows_per_w
    starts = [sum(_CHUNKS[:i]) for i in range(len(_CHUNKS))]

    mesh = plsc.VectorSubcoreMesh(core_axis_name="c", subcore_axis_name="s")

    @functools.partial(
        pl.kernel,
        out_type=jax.ShapeDtypeStruct((seq_len, d), table.dtype),
        mesh=mesh,
        scratch_types=(
            [pltpu.VMEM((c, d), table.dtype) for c in _CHUNKS]
            + [pltpu.SemaphoreType.DMA for _ in range(2 * len(_CHUNKS))]
        ),
    )
    def copy_k(table_hbm, out_hbm, *scratch):
        nb = len(_CHUNKS)
        bufs = scratch[:nb]
        gsems = scratch[nb : 2 * nb]
        ssems = scratch[2 * nb :]
        wid = lax.axis_index("c") * _NS + lax.axis_index("s")
        base = wid * rows_per_w

        gathers = [
            pltpu.make_async_copy(
                table_hbm.at[pl.ds(base + starts[i], _CHUNKS[i])],
                bufs[i],
                gsems[i],
            )
            for i in range(nb)
        ]
        scatters = [
            pltpu.make_async_copy(
                bufs[i],
                out_hbm.at[pl.ds(base + starts[i], _CHUNKS[i])],
                ssems[i],
            )
            for i in range(nb)
        ]
        for g in gathers:
            g.start()
        for i in range(nb):
            gathers[i].wait()
            scatters[i].start()
        for s in scatters:
            s.wait()

    return copy_k(table)


# speedup vs baseline: 1.0121x; 1.0121x over previous
import functools

import jax
from jax import lax
from jax.experimental import pallas as pl
from jax.experimental.pallas import tpu as pltpu
from jax.experimental.pallas import tpu_sc as plsc

_NC, _NS = 2, 16
_CHUNKS = (8, 56, 64)


def kernel(x, table):
    seq_len = x.shape[1]
    d = table.shape[1]
    nw = _NC * _NS
    rows_per_w = seq_len // nw
    assert sum(_CHUNKS) == rows_per_w
    starts = [sum(_CHUNKS[:i]) for i in range(len(_CHUNKS))]

    mesh = plsc.VectorSubcoreMesh(core_axis_name="c", subcore_axis_name="s")

    @functools.partial(
        pl.kernel,
        out_type=jax.ShapeDtypeStruct((seq_len, d), table.dtype),
        mesh=mesh,
        scratch_types=(
            [pltpu.VMEM((c, d), table.dtype) for c in _CHUNKS]
            + [pltpu.SemaphoreType.DMA for _ in range(2 * len(_CHUNKS))]
        ),
    )
    def copy_k(table_hbm, out_hbm, *scratch):
        nb = len(_CHUNKS)
        bufs = scratch[:nb]
        gsems = scratch[nb : 2 * nb]
        ssems = scratch[2 * nb :]
        wid = lax.axis_index("s") * _NC + lax.axis_index("c")
        base = wid * rows_per_w

        gathers = [
            pltpu.make_async_copy(
                table_hbm.at[pl.ds(base + starts[i], _CHUNKS[i])],
                bufs[i],
                gsems[i],
            )
            for i in range(nb)
        ]
        scatters = [
            pltpu.make_async_copy(
                bufs[i],
                out_hbm.at[pl.ds(base + starts[i], _CHUNKS[i])],
                ssems[i],
            )
            for i in range(nb)
        ]
        for g in gathers:
            g.start()
        for i in range(nb):
            gathers[i].wait()
            scatters[i].start()
        for s in scatters:
            s.wait()

    return copy_k(table)
